# baseline (speedup 1.0000x reference)
import jax
import jax.numpy as jnp
import numpy as np
from jax.experimental import pallas as pl
from jax.experimental.pallas import tpu as pltpu

N = 256
HEADS = 4
C1 = 32
OUT = 128
CH = 128
SIGNBIT16 = np.int16(-32768)


def _attend(xlT_s_h, xr_s_h, smask_h, xl_h, logits_lin, f32):
    w = xlT_s_h[None, :, :] + xr_s_h[:, :, None]
    wi = jax.lax.bitcast_convert_type(w, jnp.int16)
    wi = jnp.bitwise_and(wi, np.int16(32767))
    wi = jnp.bitwise_xor(wi, smask_h[None, :, :])
    ws = jax.lax.bitcast_convert_type(wi, jnp.bfloat16)
    c = ws.shape[1]
    s1 = ws[:, :c // 2, :] + ws[:, c // 2:, :]
    s2 = s1[:, :c // 4, :] + s1[:, c // 4:, :]
    s3 = s2[:, :c // 8, :] + s2[:, c // 8:, :]
    logits = logits_lin + jnp.sum(s3.astype(f32), axis=1)
    m = jnp.max(logits, axis=1, keepdims=True)
    e = jnp.exp(logits - m)
    alpha = e / jnp.sum(e, axis=1, keepdims=True)
    return jnp.dot(alpha, xl_h, preferred_element_type=f32)


def _gat_layer(cdim, xl, xlT, xr, att):
    bf16 = jnp.bfloat16
    cols = []
    for h in range(HEADS):
        sl = slice(h * cdim, (h + 1) * cdim)
        a_row = att[h][None, :]
        a_col = att[h][:, None]
        aval_row = 0.4 * jnp.abs(a_row)
        aval_col = 0.4 * jnp.abs(a_col)
        ai = jax.lax.bitcast_convert_type(a_col, jnp.int32)
        sm32 = jax.lax.shift_right_logical(
            jnp.bitwise_and(ai, np.int32(-2147483648)), 16)
        smask_h = sm32.astype(jnp.int16)
        A_row = 0.6 * jnp.sum(xlT[sl, :] * a_col, axis=0, keepdims=True)
        B_col = 0.6 * jnp.sum(xr[:, sl] * a_row, axis=1, keepdims=True)
        xlT_s = (xlT[sl, :] * aval_col).astype(bf16)
        xr_s = (xr[:, sl] * aval_row).astype(bf16)
        cols.append(_attend(xlT_s, xr_s, smask_h, xl[:, sl],
                            A_row + B_col, jnp.float32))
    return cols


def _gnn_body(x_ref, W1l_ref, b1l_ref, W1r_ref, b1r_ref, att1_ref,
              bias1_ref, W2l_ref, b2l_ref, W2r_ref, b2r_ref, att2_ref,
              bias2_ref, out_ref, h_scr, hT_scr):
    f32 = jnp.float32
    i = pl.program_id(0)
    nphase = pl.num_programs(0) // 2
    d0 = (i % nphase) * CH

    @pl.when(i < nphase)
    def _layer1():
        x = x_ref[...]
        xl = (jnp.dot(x, W1l_ref[...], preferred_element_type=f32)
              + b1l_ref[...])
        xlT = xl.T
        xc = x_ref[pl.ds(d0, CH), :]
        xr = (jnp.dot(xc, W1r_ref[...], preferred_element_type=f32)
              + b1r_ref[...])
        cols = _gat_layer(C1, xl, xlT, xr, att1_ref[...])
        hfeat = jnp.concatenate(cols, axis=1) + bias1_ref[...]
        hfeat = jnp.maximum(hfeat, 0.0)
        h_scr[pl.ds(d0, CH), :] = hfeat
        hT_scr[:, pl.ds(d0, CH)] = hfeat.T

    @pl.when(i >= nphase)
    def _layer2():
        hfull = h_scr[...]
        xl = (jnp.dot(hfull, W2l_ref[...], preferred_element_type=f32)
              + b2l_ref[...])
        xlT = jax.lax.dot_general(W2l_ref[...], hT_scr[...],
                                  (((0,), (0,)), ((), ())),
                                  preferred_element_type=f32)
        xlT = xlT + b2l_ref[...].T
        hc = h_scr[pl.ds(d0, CH), :]
        xr = (jnp.dot(hc, W2r_ref[...], preferred_element_type=f32)
              + b2r_ref[...])
        cols = _gat_layer(OUT, xl, xlT, xr, att2_ref[...])
        acc = cols[0] + cols[1] + cols[2] + cols[3]
        out_ref[...] = acc * (1.0 / HEADS) + bias2_ref[...]


def _full(shape):
    return pl.BlockSpec(shape, lambda i: (0,) * len(shape))


def kernel(x, W1l, b1l, W1r, b1r, att1, bias1, W2l, b2l, W2r, b2r, att2,
           bias2):
    f32 = jnp.float32
    hid = HEADS * C1
    wide = HEADS * OUT
    nphase = N // CH

    return pl.pallas_call(
        _gnn_body,
        grid=(2 * nphase,),
        in_specs=[
            _full((N, x.shape[1])),
            _full(W1l.shape), _full((1, hid)), _full(W1r.shape),
            _full((1, hid)), _full(att1.shape), _full((1, hid)),
            _full(W2l.shape), _full((1, wide)), _full(W2r.shape),
            _full((1, wide)), _full(att2.shape), _full((1, OUT)),
        ],
        out_specs=pl.BlockSpec((CH, OUT),
                               lambda i: (jnp.maximum(i - nphase, 0), 0)),
        out_shape=jax.ShapeDtypeStruct((N, OUT), f32),
        scratch_shapes=[pltpu.VMEM((N, hid), f32),
                        pltpu.VMEM((hid, N), f32)],
    )(x, W1l, b1l.reshape(1, -1), W1r, b1r.reshape(1, -1), att1,
      bias1.reshape(1, -1), W2l, b2l.reshape(1, -1), W2r,
      b2r.reshape(1, -1), att2, bias2.reshape(1, -1))


# speedup vs baseline: 1.0020x; 1.0020x over previous
import jax
import jax.numpy as jnp
import numpy as np
from jax.experimental import pallas as pl
from jax.experimental.pallas import tpu as pltpu

N = 256
HEADS = 4
C1 = 32
OUT = 128
CH = 128
SIGNBIT16 = np.int16(-32768)


def _attend(xlT_s_h, xr_s_h, smask_h, xl_h, logits_lin, f32):
    w = xlT_s_h[None, :, :] + xr_s_h[:, :, None]
    wi = jax.lax.bitcast_convert_type(w, jnp.int16)
    wi = jnp.bitwise_and(wi, np.int16(32767))
    wi = jnp.bitwise_xor(wi, smask_h[None, :, :])
    ws = jax.lax.bitcast_convert_type(wi, jnp.bfloat16)
    c = ws.shape[1]
    s1 = ws[:, :c // 2, :] + ws[:, c // 2:, :]
    s2 = s1[:, :c // 4, :] + s1[:, c // 4:, :]
    logits = logits_lin + jnp.sum(s2.astype(f32), axis=1)
    m = jnp.max(logits, axis=1, keepdims=True)
    e = jnp.exp(logits - m)
    alpha = e / jnp.sum(e, axis=1, keepdims=True)
    return jnp.dot(alpha, xl_h, preferred_element_type=f32)


def _gat_layer(cdim, xl, xlT, xr, att):
    bf16 = jnp.bfloat16
    cols = []
    for h in range(HEADS):
        sl = slice(h * cdim, (h + 1) * cdim)
        a_row = att[h][None, :]
        a_col = att[h][:, None]
        aval_row = 0.4 * jnp.abs(a_row)
        aval_col = 0.4 * jnp.abs(a_col)
        ai = jax.lax.bitcast_convert_type(a_col, jnp.int32)
        sm32 = jax.lax.shift_right_logical(
            jnp.bitwise_and(ai, np.int32(-2147483648)), 16)
        smask_h = sm32.astype(jnp.int16)
        A_row = 0.6 * jnp.sum(xlT[sl, :] * a_col, axis=0, keepdims=True)
        B_col = 0.6 * jnp.sum(xr[:, sl] * a_row, axis=1, keepdims=True)
        xlT_s = (xlT[sl, :] * aval_col).astype(bf16)
        xr_s = (xr[:, sl] * aval_row).astype(bf16)
        cols.append(_attend(xlT_s, xr_s, smask_h, xl[:, sl],
                            A_row + B_col, jnp.float32))
    return cols


def _gnn_body(x_ref, W1l_ref, b1l_ref, W1r_ref, b1r_ref, att1_ref,
              bias1_ref, W2l_ref, b2l_ref, W2r_ref, b2r_ref, att2_ref,
              bias2_ref, out_ref, h_scr, hT_scr):
    f32 = jnp.float32
    i = pl.program_id(0)
    nphase = pl.num_programs(0) // 2
    d0 = (i % nphase) * CH

    @pl.when(i < nphase)
    def _layer1():
        x = x_ref[...]
        xl = (jnp.dot(x, W1l_ref[...], preferred_element_type=f32)
              + b1l_ref[...])
        xlT = xl.T
        xc = x_ref[pl.ds(d0, CH), :]
        xr = (jnp.dot(xc, W1r_ref[...], preferred_element_type=f32)
              + b1r_ref[...])
        cols = _gat_layer(C1, xl, xlT, xr, att1_ref[...])
        hfeat = jnp.concatenate(cols, axis=1) + bias1_ref[...]
        hfeat = jnp.maximum(hfeat, 0.0)
        h_scr[pl.ds(d0, CH), :] = hfeat
        hT_scr[:, pl.ds(d0, CH)] = hfeat.T

    @pl.when(i >= nphase)
    def _layer2():
        hfull = h_scr[...]
        xl = (jnp.dot(hfull, W2l_ref[...], preferred_element_type=f32)
              + b2l_ref[...])
        xlT = jax.lax.dot_general(W2l_ref[...], hT_scr[...],
                                  (((0,), (0,)), ((), ())),
                                  preferred_element_type=f32)
        xlT = xlT + b2l_ref[...].T
        hc = h_scr[pl.ds(d0, CH), :]
        xr = (jnp.dot(hc, W2r_ref[...], preferred_element_type=f32)
              + b2r_ref[...])
        cols = _gat_layer(OUT, xl, xlT, xr, att2_ref[...])
        acc = cols[0] + cols[1] + cols[2] + cols[3]
        out_ref[...] = acc * (1.0 / HEADS) + bias2_ref[...]


def _full(shape):
    return pl.BlockSpec(shape, lambda i: (0,) * len(shape))


def kernel(x, W1l, b1l, W1r, b1r, att1, bias1, W2l, b2l, W2r, b2r, att2,
           bias2):
    f32 = jnp.float32
    hid = HEADS * C1
    wide = HEADS * OUT
    nphase = N // CH

    return pl.pallas_call(
        _gnn_body,
        grid=(2 * nphase,),
        in_specs=[
            _full((N, x.shape[1])),
            _full(W1l.shape), _full((1, hid)), _full(W1r.shape),
            _full((1, hid)), _full(att1.shape), _full((1, hid)),
            _full(W2l.shape), _full((1, wide)), _full(W2r.shape),
            _full((1, wide)), _full(att2.shape), _full((1, OUT)),
        ],
        out_specs=pl.BlockSpec((CH, OUT),
                               lambda i: (jnp.maximum(i - nphase, 0), 0)),
        out_shape=jax.ShapeDtypeStruct((N, OUT), f32),
        scratch_shapes=[pltpu.VMEM((N, hid), f32),
                        pltpu.VMEM((hid, N), f32)],
    )(x, W1l, b1l.reshape(1, -1), W1r, b1r.reshape(1, -1), att1,
      bias1.reshape(1, -1), W2l, b2l.reshape(1, -1), W2r,
      b2r.reshape(1, -1), att2, bias2.reshape(1, -1))
